# baseline (speedup 1.0000x reference)
import jax
import jax.numpy as jnp
from jax import lax
from jax.experimental import pallas as pl
from jax.experimental.pallas import tpu as pltpu
from jax.experimental.pallas import tpu_sc as plsc

NUM_USERS = 25000
NUM_ITEMS = 25000
N_NODES = NUM_USERS + NUM_ITEMS
N_EDGES = 800000
N_LAYERS = 3

NC = 2
NS = 16
LANES = 16
NGROUPS = 4
GW = 32

CHUNK = 384
CHUNKS_PER_TILE = 131
EPAD = NS * CHUNKS_PER_TILE * CHUNK
EROWS = EPAD // 128
ROWS_PER_TILE = EROWS // NS
NPAD = 50176
NODES_PER_TILE = NPAD // NS


def _mm_body(a_ref, w_ref, o_ref):
    o_ref[...] = jnp.dot(a_ref[...], w_ref[...],
                         preferred_element_type=jnp.float32)


def _attr_matmul(a, wt):
    return pl.pallas_call(
        _mm_body,
        out_shape=jax.ShapeDtypeStruct((a.shape[0], wt.shape[1]), jnp.float32),
    )(a, wt)


def _prop_body(emb_in, src4_hbm, dst_hbm, w_hbm, emb_out, emb_scr,
               acc, src_v, dst_v, w_v, rows_v, rows_o, sem_l, sem_s):
    c = lax.axis_index("c")
    s = lax.axis_index("s")
    lanes = lax.broadcasted_iota(jnp.int32, (LANES,), 0)

    def run_layer(g, src_tab, dst_tab):
        def chunk_body(ci, carry):
            r0 = s * ROWS_PER_TILE + ci * (CHUNK // 128)
            d1 = pltpu.async_copy(src4_hbm.at[g, pl.ds(r0, CHUNK // 128)],
                                  src_v, sem_l)
            d2 = pltpu.async_copy(dst_hbm.at[pl.ds(r0, CHUNK // 128)],
                                  dst_v, sem_l)
            d3 = pltpu.async_copy(w_hbm.at[pl.ds(r0 * 128, CHUNK)], w_v, sem_l)
            d1.wait(); d2.wait(); d3.wait()
            gds = [pltpu.async_copy(src_tab.at[src_v.at[j]],
                                    rows_v.at[pl.ds(j * 128, 128)], sem_l)
                   for j in range(CHUNK // 128)]
            for d in gds:
                d.wait()

            def scale_body(k, carry2):
                e0 = k * LANES
                wv = w_v[pl.ds(e0, LANES)]
                eidx = e0 + lanes
                for col in range(GW):
                    cidx = jnp.full((LANES,), col, jnp.int32)
                    vals = plsc.load_gather(rows_v, [eidx, cidx])
                    plsc.store_scatter(rows_o, [eidx, cidx], vals * wv)
                return carry2

            lax.fori_loop(0, CHUNK // LANES, scale_body, 0)

            sds = [pltpu.async_copy(rows_o.at[pl.ds(j * 128, 128)],
                                    acc.at[dst_v.at[j]], sem_s, add=True)
                   for j in range(CHUNK // 128)]
            for d in sds:
                d.wait()
            return carry

        lax.fori_loop(0, CHUNKS_PER_TILE, chunk_body, 0)
        plsc.subcore_barrier()
        pltpu.sync_copy(
            acc.at[pl.ds(s * NODES_PER_TILE, NODES_PER_TILE)],
            dst_tab.at[pl.ds(g * NPAD + s * NODES_PER_TILE,
                             NODES_PER_TILE)])
        plsc.subcore_barrier()

    for p in range(NGROUPS // NC):
        g = c * (NGROUPS // NC) + p
        pltpu.sync_copy(
            emb_in.at[pl.ds(g * NPAD + s * NODES_PER_TILE,
                            NODES_PER_TILE)],
            acc.at[pl.ds(s * NODES_PER_TILE, NODES_PER_TILE)])
        plsc.subcore_barrier()
        run_layer(g, emb_in, emb_out)
        run_layer(g, emb_out, emb_scr)
        run_layer(g, emb_scr, emb_out)


@jax.jit
def _propagate(emb4, src4, dst2d, w1d):
    mesh = plsc.VectorSubcoreMesh(core_axis_name="c", subcore_axis_name="s")
    f = pl.kernel(
        _prop_body,
        out_type=(
            jax.ShapeDtypeStruct((NGROUPS * NPAD, GW), jnp.float32),
            jax.ShapeDtypeStruct((NGROUPS * NPAD, GW), jnp.float32),
        ),
        mesh=mesh,
        compiler_params=pltpu.CompilerParams(
            needs_layout_passes=False, use_tc_tiling_on_sc=False),
        scratch_types=[
            pltpu.VMEM_SHARED((NPAD, GW), jnp.float32),
            pltpu.VMEM((CHUNK // 128, 128), jnp.int32),
            pltpu.VMEM((CHUNK // 128, 128), jnp.int32),
            pltpu.VMEM((CHUNK,), jnp.float32),
            pltpu.VMEM((CHUNK, GW), jnp.float32),
            pltpu.VMEM((CHUNK, GW), jnp.float32),
            pltpu.SemaphoreType.DMA,
            pltpu.SemaphoreType.DMA,
        ],
    )
    return f(emb4, src4, dst2d, w1d)


def kernel(missing_attr, user_emb, item_emb, trans_w, edge_weight, edge_index):
    attr = _attr_matmul(missing_attr, trans_w.T)
    emb = jnp.concatenate(
        [user_emb, jnp.concatenate([item_emb, attr], axis=1)], axis=0)
    emb = jnp.pad(emb, ((0, NPAD - N_NODES), (0, 0)))
    emb4 = emb.reshape(NPAD, NGROUPS, GW).transpose(1, 0, 2)
    emb4 = emb4.reshape(NGROUPS * NPAD, GW)

    pad = EPAD - N_EDGES
    src = jnp.concatenate([edge_index[0], jnp.zeros((pad,), jnp.int32)])
    dst = jnp.concatenate([edge_index[1], jnp.zeros((pad,), jnp.int32)])
    w = jnp.concatenate([edge_weight, jnp.zeros((pad,), jnp.float32)])
    goff = (jnp.arange(NGROUPS, dtype=jnp.int32) * NPAD)[:, None]
    src4 = (src[None, :] + goff).reshape(NGROUPS, EROWS, 128)
    dst2d = dst.reshape(EROWS, 128)

    out, _ = _propagate(emb4, src4, dst2d, w)
    final = out.reshape(NGROUPS, NPAD, GW).transpose(1, 0, 2)
    final = final.reshape(NPAD, NGROUPS * GW)
    return final[:NUM_USERS], final[NUM_USERS:N_NODES]


# speedup vs baseline: 1.3735x; 1.3735x over previous
import jax
import jax.numpy as jnp
from jax import lax
from jax.experimental import pallas as pl
from jax.experimental.pallas import tpu as pltpu
from jax.experimental.pallas import tpu_sc as plsc

NUM_USERS = 25000
NUM_ITEMS = 25000
N_NODES = NUM_USERS + NUM_ITEMS
N_EDGES = 800000
N_LAYERS = 3

NC = 2
NS = 16
LANES = 16
NGROUPS = 4
GW = 32

CHUNK = 384
CHUNKS_PER_TILE = 131
EPAD = NS * CHUNKS_PER_TILE * CHUNK
EROWS = EPAD // 128
ROWS_PER_TILE = EROWS // NS
NPAD = 50176
NODES_PER_TILE = NPAD // NS


def _mm_body(a_ref, w_ref, o_ref):
    o_ref[...] = jnp.dot(a_ref[...], w_ref[...],
                         preferred_element_type=jnp.float32)


def _attr_matmul(a, wt):
    return pl.pallas_call(
        _mm_body,
        out_shape=jax.ShapeDtypeStruct((a.shape[0], wt.shape[1]), jnp.float32),
    )(a, wt)


def _prop_body(emb_in, src4_hbm, dst_hbm, w_hbm, emb_out, emb_scr,
               acc, src_v, dst_v, w_v, rows_v, rows_o, sem_l, sem_s):
    c = lax.axis_index("c")
    s = lax.axis_index("s")
    lanes = lax.broadcasted_iota(jnp.int32, (LANES,), 0)

    def run_layer(g, src_tab, dst_tab):
        def chunk_body(ci, carry):
            r0 = s * ROWS_PER_TILE + ci * (CHUNK // 128)
            d1 = pltpu.async_copy(src4_hbm.at[g, pl.ds(r0, CHUNK // 128)],
                                  src_v, sem_l)
            d2 = pltpu.async_copy(dst_hbm.at[pl.ds(r0, CHUNK // 128)],
                                  dst_v, sem_l)
            d3 = pltpu.async_copy(w_hbm.at[pl.ds(r0 * 128, CHUNK)], w_v, sem_l)
            d1.wait(); d2.wait(); d3.wait()
            gds = [pltpu.async_copy(src_tab.at[src_v.at[j]],
                                    rows_v.at[pl.ds(j * 128, 128)], sem_l)
                   for j in range(CHUNK // 128)]
            for d in gds:
                d.wait()

            def scale_body(k, carry2):
                e0 = k * LANES
                wv = w_v[pl.ds(e0, LANES)]
                eidx = e0 + lanes
                cidxs = [jnp.full((LANES,), col, jnp.int32)
                         for col in range(GW)]
                vals = [plsc.load_gather(rows_v, [eidx, cidxs[col]])
                        for col in range(GW)]
                for col in range(GW):
                    plsc.store_scatter(rows_o, [eidx, cidxs[col]],
                                       vals[col] * wv)
                return carry2

            lax.fori_loop(0, CHUNK // LANES, scale_body, 0)

            sds = [pltpu.async_copy(rows_o.at[pl.ds(j * 128, 128)],
                                    acc.at[dst_v.at[j]], sem_s, add=True)
                   for j in range(CHUNK // 128)]
            for d in sds:
                d.wait()
            return carry

        lax.fori_loop(0, CHUNKS_PER_TILE, chunk_body, 0)
        plsc.subcore_barrier()
        pltpu.sync_copy(
            acc.at[pl.ds(s * NODES_PER_TILE, NODES_PER_TILE)],
            dst_tab.at[pl.ds(g * NPAD + s * NODES_PER_TILE,
                             NODES_PER_TILE)])
        plsc.subcore_barrier()

    for p in range(NGROUPS // NC):
        g = c * (NGROUPS // NC) + p
        pltpu.sync_copy(
            emb_in.at[pl.ds(g * NPAD + s * NODES_PER_TILE,
                            NODES_PER_TILE)],
            acc.at[pl.ds(s * NODES_PER_TILE, NODES_PER_TILE)])
        plsc.subcore_barrier()
        run_layer(g, emb_in, emb_out)
        run_layer(g, emb_out, emb_scr)
        run_layer(g, emb_scr, emb_out)


@jax.jit
def _propagate(emb4, src4, dst2d, w1d):
    mesh = plsc.VectorSubcoreMesh(core_axis_name="c", subcore_axis_name="s")
    f = pl.kernel(
        _prop_body,
        out_type=(
            jax.ShapeDtypeStruct((NGROUPS * NPAD, GW), jnp.float32),
            jax.ShapeDtypeStruct((NGROUPS * NPAD, GW), jnp.float32),
        ),
        mesh=mesh,
        compiler_params=pltpu.CompilerParams(
            needs_layout_passes=False, use_tc_tiling_on_sc=False),
        scratch_types=[
            pltpu.VMEM_SHARED((NPAD, GW), jnp.float32),
            pltpu.VMEM((CHUNK // 128, 128), jnp.int32),
            pltpu.VMEM((CHUNK // 128, 128), jnp.int32),
            pltpu.VMEM((CHUNK,), jnp.float32),
            pltpu.VMEM((CHUNK, GW), jnp.float32),
            pltpu.VMEM((CHUNK, GW), jnp.float32),
            pltpu.SemaphoreType.DMA,
            pltpu.SemaphoreType.DMA,
        ],
    )
    return f(emb4, src4, dst2d, w1d)


def kernel(missing_attr, user_emb, item_emb, trans_w, edge_weight, edge_index):
    attr = _attr_matmul(missing_attr, trans_w.T)
    emb = jnp.concatenate(
        [user_emb, jnp.concatenate([item_emb, attr], axis=1)], axis=0)
    emb = jnp.pad(emb, ((0, NPAD - N_NODES), (0, 0)))
    emb4 = emb.reshape(NPAD, NGROUPS, GW).transpose(1, 0, 2)
    emb4 = emb4.reshape(NGROUPS * NPAD, GW)

    pad = EPAD - N_EDGES
    src = jnp.concatenate([edge_index[0], jnp.zeros((pad,), jnp.int32)])
    dst = jnp.concatenate([edge_index[1], jnp.zeros((pad,), jnp.int32)])
    w = jnp.concatenate([edge_weight, jnp.zeros((pad,), jnp.float32)])
    goff = (jnp.arange(NGROUPS, dtype=jnp.int32) * NPAD)[:, None]
    src4 = (src[None, :] + goff).reshape(NGROUPS, EROWS, 128)
    dst2d = dst.reshape(EROWS, 128)

    out, _ = _propagate(emb4, src4, dst2d, w)
    final = out.reshape(NGROUPS, NPAD, GW).transpose(1, 0, 2)
    final = final.reshape(NPAD, NGROUPS * GW)
    return final[:NUM_USERS], final[NUM_USERS:N_NODES]


# speedup vs baseline: 4.8717x; 3.5469x over previous
import jax
import jax.numpy as jnp
from jax import lax
from jax.experimental import pallas as pl
from jax.experimental.pallas import tpu as pltpu
from jax.experimental.pallas import tpu_sc as plsc

NUM_USERS = 25000
NUM_ITEMS = 25000
N_NODES = NUM_USERS + NUM_ITEMS
N_EDGES = 800000
N_LAYERS = 3

NC = 2
NS = 16
LANES = 16
NGROUPS = 4
GW = 32

CHUNK = 384
CHUNKS_PER_TILE = 131
EPAD = NS * CHUNKS_PER_TILE * CHUNK
EROWS = EPAD // 128
ROWS_PER_TILE = EROWS // NS
NPAD = 50176
NODES_PER_TILE = NPAD // NS


def _mm_body(a_ref, w_ref, o_ref):
    o_ref[...] = jnp.dot(a_ref[...], w_ref[...],
                         preferred_element_type=jnp.float32)


def _attr_matmul(a, wt):
    return pl.pallas_call(
        _mm_body,
        out_shape=jax.ShapeDtypeStruct((a.shape[0], wt.shape[1]), jnp.float32),
    )(a, wt)


def _prop_body(emb_in, src4_hbm, dst_hbm, w_hbm, emb_out, emb_scr,
               acc, src_v, dst_v, w_v, rows_v, rows_o, sem_l, sem_s):
    c = lax.axis_index("c")
    s = lax.axis_index("s")

    def run_layer(g, src_tab, dst_tab):
        def chunk_body(ci, carry):
            r0 = s * ROWS_PER_TILE + ci * (CHUNK // 128)
            d1 = pltpu.async_copy(src4_hbm.at[g, pl.ds(r0, CHUNK // 128)],
                                  src_v, sem_l)
            d2 = pltpu.async_copy(dst_hbm.at[pl.ds(r0, CHUNK // 128)],
                                  dst_v, sem_l)
            d3 = pltpu.async_copy(w_hbm.at[pl.ds(r0 * 128, CHUNK)], w_v, sem_l)
            d1.wait(); d2.wait(); d3.wait()
            gds = [pltpu.async_copy(src_tab.at[src_v.at[j]],
                                    rows_v.at[pl.ds(j * 128, 128)], sem_l)
                   for j in range(CHUNK // 128)]
            for d in gds:
                d.wait()

            def scale_body(k, carry2):
                e0 = k * LANES
                wv = w_v[pl.ds(e0, LANES)]
                vals = []
                for i in range(LANES):
                    vals.append(rows_v[e0 + i, pl.ds(0, LANES)])
                    vals.append(rows_v[e0 + i, pl.ds(LANES, LANES)])
                for i in range(LANES):
                    ws = wv[i]
                    rows_o[e0 + i, pl.ds(0, LANES)] = vals[2 * i] * ws
                    rows_o[e0 + i, pl.ds(LANES, LANES)] = vals[2 * i + 1] * ws
                return carry2

            lax.fori_loop(0, CHUNK // LANES, scale_body, 0)

            sds = [pltpu.async_copy(rows_o.at[pl.ds(j * 128, 128)],
                                    acc.at[dst_v.at[j]], sem_s, add=True)
                   for j in range(CHUNK // 128)]
            for d in sds:
                d.wait()
            return carry

        lax.fori_loop(0, CHUNKS_PER_TILE, chunk_body, 0)
        plsc.subcore_barrier()
        pltpu.sync_copy(
            acc.at[pl.ds(s * NODES_PER_TILE, NODES_PER_TILE)],
            dst_tab.at[pl.ds(g * NPAD + s * NODES_PER_TILE,
                             NODES_PER_TILE)])
        plsc.subcore_barrier()

    for p in range(NGROUPS // NC):
        g = c * (NGROUPS // NC) + p
        pltpu.sync_copy(
            emb_in.at[pl.ds(g * NPAD + s * NODES_PER_TILE,
                            NODES_PER_TILE)],
            acc.at[pl.ds(s * NODES_PER_TILE, NODES_PER_TILE)])
        plsc.subcore_barrier()
        run_layer(g, emb_in, emb_out)
        run_layer(g, emb_out, emb_scr)
        run_layer(g, emb_scr, emb_out)


@jax.jit
def _propagate(emb4, src4, dst2d, w1d):
    mesh = plsc.VectorSubcoreMesh(core_axis_name="c", subcore_axis_name="s")
    f = pl.kernel(
        _prop_body,
        out_type=(
            jax.ShapeDtypeStruct((NGROUPS * NPAD, GW), jnp.float32),
            jax.ShapeDtypeStruct((NGROUPS * NPAD, GW), jnp.float32),
        ),
        mesh=mesh,
        compiler_params=pltpu.CompilerParams(
            needs_layout_passes=False, use_tc_tiling_on_sc=False),
        scratch_types=[
            pltpu.VMEM_SHARED((NPAD, GW), jnp.float32),
            pltpu.VMEM((CHUNK // 128, 128), jnp.int32),
            pltpu.VMEM((CHUNK // 128, 128), jnp.int32),
            pltpu.VMEM((CHUNK,), jnp.float32),
            pltpu.VMEM((CHUNK, GW), jnp.float32),
            pltpu.VMEM((CHUNK, GW), jnp.float32),
            pltpu.SemaphoreType.DMA,
            pltpu.SemaphoreType.DMA,
        ],
    )
    return f(emb4, src4, dst2d, w1d)


def kernel(missing_attr, user_emb, item_emb, trans_w, edge_weight, edge_index):
    attr = _attr_matmul(missing_attr, trans_w.T)
    emb = jnp.concatenate(
        [user_emb, jnp.concatenate([item_emb, attr], axis=1)], axis=0)
    emb = jnp.pad(emb, ((0, NPAD - N_NODES), (0, 0)))
    emb4 = emb.reshape(NPAD, NGROUPS, GW).transpose(1, 0, 2)
    emb4 = emb4.reshape(NGROUPS * NPAD, GW)

    pad = EPAD - N_EDGES
    src = jnp.concatenate([edge_index[0], jnp.zeros((pad,), jnp.int32)])
    dst = jnp.concatenate([edge_index[1], jnp.zeros((pad,), jnp.int32)])
    w = jnp.concatenate([edge_weight, jnp.zeros((pad,), jnp.float32)])
    goff = (jnp.arange(NGROUPS, dtype=jnp.int32) * NPAD)[:, None]
    src4 = (src[None, :] + goff).reshape(NGROUPS, EROWS, 128)
    dst2d = dst.reshape(EROWS, 128)

    out, _ = _propagate(emb4, src4, dst2d, w)
    final = out.reshape(NGROUPS, NPAD, GW).transpose(1, 0, 2)
    final = final.reshape(NPAD, NGROUPS * GW)
    return final[:NUM_USERS], final[NUM_USERS:N_NODES]


# speedup vs baseline: 7.9521x; 1.6323x over previous
import jax
import jax.numpy as jnp
from jax import lax
from jax.experimental import pallas as pl
from jax.experimental.pallas import tpu as pltpu
from jax.experimental.pallas import tpu_sc as plsc

NUM_USERS = 25000
NUM_ITEMS = 25000
N_NODES = NUM_USERS + NUM_ITEMS
N_EDGES = 800000
N_LAYERS = 3

NC = 2
NS = 16
LANES = 16
NGROUPS = 4
GW = 32

CHUNK = 384
CHUNKS_PER_TILE = 131
EPAD = NS * CHUNKS_PER_TILE * CHUNK
EROWS = EPAD // 128
ROWS_PER_TILE = EROWS // NS
NPAD = 50176
NODES_PER_TILE = NPAD // NS


def _mm_body(a_ref, w_ref, o_ref):
    o_ref[...] = jnp.dot(a_ref[...], w_ref[...],
                         preferred_element_type=jnp.float32)


def _attr_matmul(a, wt):
    return pl.pallas_call(
        _mm_body,
        out_shape=jax.ShapeDtypeStruct((a.shape[0], wt.shape[1]), jnp.float32),
    )(a, wt)


def _prop_body(emb_in, src4_hbm, dst_hbm, w_hbm, emb_out, emb_scr,
               acc, src_v, dst_v, w_v, rows_v, sem_i, sem_g, sem_s):
    c = lax.axis_index("c")
    s = lax.axis_index("s")
    RPC = CHUNK // 128

    def run_layer(g, src_tab, dst_tab):
        def idx_pairs(ci):
            b2 = lax.rem(ci, 2)
            b3 = lax.rem(ci, 3)
            r0 = s * ROWS_PER_TILE + ci * RPC
            return [
                (src4_hbm.at[g, pl.ds(r0, RPC)], src_v.at[b2]),
                (dst_hbm.at[pl.ds(r0, RPC)], dst_v.at[b3]),
                (w_hbm.at[pl.ds(r0 * 128, CHUNK)], w_v.at[b2]),
            ]

        def fire_idx(ci):
            for a, d in idx_pairs(ci):
                pltpu.async_copy(a, d, sem_i)

        def wait_idx(ci):
            for a, d in idx_pairs(ci):
                pltpu.make_async_copy(a, d, sem_i).wait()

        def gather_pairs(ci):
            b2 = lax.rem(ci, 2)
            return [(src_tab.at[src_v.at[b2, j]],
                     rows_v.at[b2, pl.ds(j * 128, 128)]) for j in range(RPC)]

        def fire_gather(ci):
            for a, d in gather_pairs(ci):
                pltpu.async_copy(a, d, sem_g)

        def wait_gather(ci):
            for a, d in gather_pairs(ci):
                pltpu.make_async_copy(a, d, sem_g).wait()

        def scatter_pairs(ci):
            b2 = lax.rem(ci, 2)
            b3 = lax.rem(ci, 3)
            return [(rows_v.at[b2, pl.ds(j * 128, 128)],
                     acc.at[dst_v.at[b3, j]]) for j in range(RPC)]

        def fire_scatter(ci):
            for a, d in scatter_pairs(ci):
                pltpu.async_copy(a, d, sem_s, add=True)

        def wait_scatter(ci):
            for a, d in scatter_pairs(ci):
                pltpu.make_async_copy(a, d, sem_s).wait()

        def compute(ci):
            b2 = lax.rem(ci, 2)

            def scale_body(k, carry2):
                e0 = k * LANES
                wv = w_v[b2, pl.ds(e0, LANES)]
                vals = []
                for i in range(LANES):
                    vals.append(rows_v[b2, e0 + i, pl.ds(0, LANES)])
                    vals.append(rows_v[b2, e0 + i, pl.ds(LANES, LANES)])
                for i in range(LANES):
                    ws = wv[i]
                    rows_v[b2, e0 + i, pl.ds(0, LANES)] = vals[2 * i] * ws
                    rows_v[b2, e0 + i, pl.ds(LANES, LANES)] = (
                        vals[2 * i + 1] * ws)
                return carry2

            lax.fori_loop(0, CHUNK // LANES, scale_body, 0)

        fire_idx(0)
        fire_idx(1)
        wait_idx(0)
        fire_gather(0)

        def pipe_body(ci, carry):
            wait_gather(ci)

            @pl.when(ci > 0)
            def _():
                wait_scatter(ci - 1)

            @pl.when(ci < CHUNKS_PER_TILE - 1)
            def _():
                wait_idx(ci + 1)
                fire_gather(ci + 1)

            compute(ci)
            fire_scatter(ci)

            @pl.when(ci < CHUNKS_PER_TILE - 2)
            def _():
                fire_idx(ci + 2)

            return carry

        lax.fori_loop(0, CHUNKS_PER_TILE, pipe_body, 0)
        wait_scatter(CHUNKS_PER_TILE - 1)
        plsc.subcore_barrier()
        pltpu.sync_copy(
            acc.at[pl.ds(s * NODES_PER_TILE, NODES_PER_TILE)],
            dst_tab.at[pl.ds(g * NPAD + s * NODES_PER_TILE,
                             NODES_PER_TILE)])
        plsc.subcore_barrier()

    for p in range(NGROUPS // NC):
        g = c * (NGROUPS // NC) + p
        pltpu.sync_copy(
            emb_in.at[pl.ds(g * NPAD + s * NODES_PER_TILE,
                            NODES_PER_TILE)],
            acc.at[pl.ds(s * NODES_PER_TILE, NODES_PER_TILE)])
        plsc.subcore_barrier()
        run_layer(g, emb_in, emb_out)
        run_layer(g, emb_out, emb_scr)
        run_layer(g, emb_scr, emb_out)


@jax.jit
def _propagate(emb4, src4, dst2d, w1d):
    mesh = plsc.VectorSubcoreMesh(core_axis_name="c", subcore_axis_name="s")
    f = pl.kernel(
        _prop_body,
        out_type=(
            jax.ShapeDtypeStruct((NGROUPS * NPAD, GW), jnp.float32),
            jax.ShapeDtypeStruct((NGROUPS * NPAD, GW), jnp.float32),
        ),
        mesh=mesh,
        compiler_params=pltpu.CompilerParams(
            needs_layout_passes=False, use_tc_tiling_on_sc=False),
        scratch_types=[
            pltpu.VMEM_SHARED((NPAD, GW), jnp.float32),
            pltpu.VMEM((2, CHUNK // 128, 128), jnp.int32),
            pltpu.VMEM((3, CHUNK // 128, 128), jnp.int32),
            pltpu.VMEM((2, CHUNK), jnp.float32),
            pltpu.VMEM((2, CHUNK, GW), jnp.float32),
            pltpu.SemaphoreType.DMA,
            pltpu.SemaphoreType.DMA,
            pltpu.SemaphoreType.DMA,
        ],
    )
    return f(emb4, src4, dst2d, w1d)


def kernel(missing_attr, user_emb, item_emb, trans_w, edge_weight, edge_index):
    attr = _attr_matmul(missing_attr, trans_w.T)
    emb = jnp.concatenate(
        [user_emb, jnp.concatenate([item_emb, attr], axis=1)], axis=0)
    emb = jnp.pad(emb, ((0, NPAD - N_NODES), (0, 0)))
    emb4 = emb.reshape(NPAD, NGROUPS, GW).transpose(1, 0, 2)
    emb4 = emb4.reshape(NGROUPS * NPAD, GW)

    pad = EPAD - N_EDGES
    src = jnp.concatenate([edge_index[0], jnp.zeros((pad,), jnp.int32)])
    dst = jnp.concatenate([edge_index[1], jnp.zeros((pad,), jnp.int32)])
    w = jnp.concatenate([edge_weight, jnp.zeros((pad,), jnp.float32)])
    goff = (jnp.arange(NGROUPS, dtype=jnp.int32) * NPAD)[:, None]
    src4 = (src[None, :] + goff).reshape(NGROUPS, EROWS, 128)
    dst2d = dst.reshape(EROWS, 128)

    out, _ = _propagate(emb4, src4, dst2d, w)
    final = out.reshape(NGROUPS, NPAD, GW).transpose(1, 0, 2)
    final = final.reshape(NPAD, NGROUPS * GW)
    return final[:NUM_USERS], final[NUM_USERS:N_NODES]
